# baseline (speedup 1.0000x reference)
import functools

import jax
import jax.numpy as jnp
from jax import lax
from jax.experimental import pallas as pl
from jax.experimental.pallas import tpu as pltpu
from jax.experimental.pallas import tpu_sc as plsc

_N = 1000
_D0 = 4096
_D1 = 26
_NW = 32
_EPW = _D0 // _NW
_NB = 4

_mesh = plsc.VectorSubcoreMesh(core_axis_name="c", subcore_axis_name="s")


@functools.partial(
    pl.kernel,
    mesh=_mesh,
    out_type=jax.ShapeDtypeStruct((_D0, _D1, _N), jnp.float32),
    scratch_types=[
        pltpu.VMEM((_EPW * _D1 + 16,), jnp.int32),
        pltpu.VMEM((_NB, _D1, _N), jnp.float32),
        pltpu.SemaphoreType.DMA((_NB,)),
    ],
    compiler_params=pltpu.CompilerParams(
        use_tc_tiling_on_sc=False,
        needs_layout_passes=False,
    ),
)
def _sc_onehot(idx_hbm, out_hbm, idx_v, buf, sems):
    wid = lax.axis_index("c") * 16 + lax.axis_index("s")
    lanes = lax.iota(jnp.int32, 16)
    ones = jnp.full((16,), 1.0, jnp.float32)
    zeros = jnp.zeros((16,), jnp.float32)
    tail_mask = lanes < (_D1 - 16)

    pltpu.sync_copy(
        idx_hbm.at[pl.ds(wid * _EPW * _D1, _EPW * _D1)],
        idx_v.at[pl.ds(0, _EPW * _D1)],
    )

    def _zero_row(r, _):
        s = r // _D1
        rr = r - s * _D1

        def _zero_vec(j, _):
            buf[s, rr, pl.ds(j * 16, 16)] = zeros
            return 0

        lax.fori_loop(0, _N // 16, _zero_vec, 0)
        buf[s, rr, pl.ds(_N - 16, 16)] = zeros
        return 0

    lax.fori_loop(0, _NB * _D1, _zero_row, 0)

    def _positions(c):
        base = c * _D1
        cols0 = idx_v[pl.ds(base, 16)]
        cols1 = idx_v[pl.ds(base + 16, 16)]
        return cols0, cols1

    def _chunk(c, _):
        s = lax.rem(c, _NB)
        d0 = wid * _EPW + c

        @pl.when(c >= _NB)
        def _recycle():
            pltpu.make_async_copy(
                buf.at[s], out_hbm.at[d0 - _NB], sems.at[s]
            ).wait()
            oc0, oc1 = _positions(c - _NB)
            plsc.store_scatter(buf.at[s], [lanes, oc0], zeros)
            plsc.store_scatter(
                buf.at[s], [16 + lanes, oc1], zeros, mask=tail_mask
            )

        cols0, cols1 = _positions(c)
        plsc.store_scatter(buf.at[s], [lanes, cols0], ones)
        plsc.store_scatter(
            buf.at[s], [16 + lanes, cols1], ones, mask=tail_mask
        )
        pltpu.make_async_copy(buf.at[s], out_hbm.at[d0], sems.at[s]).start()
        return 0

    lax.fori_loop(0, _EPW, _chunk, 0)

    for k in range(_NB):
        pltpu.make_async_copy(
            buf.at[k], out_hbm.at[wid * _EPW + _EPW - _NB + k], sems.at[k]
        ).wait()


def kernel(input):
    return _sc_onehot(input.reshape(-1))


# speedup vs baseline: 1.9707x; 1.9707x over previous
import functools

import jax
import jax.numpy as jnp
from jax import lax
from jax.experimental import pallas as pl
from jax.experimental.pallas import tpu as pltpu
from jax.experimental.pallas import tpu_sc as plsc

_N = 1000
_D0 = 4096
_D1 = 26
_NW = 32
_EPW = _D0 // _NW
_NB = 2

_mesh = plsc.VectorSubcoreMesh(core_axis_name="c", subcore_axis_name="s")


@functools.partial(
    pl.kernel,
    mesh=_mesh,
    out_type=jax.ShapeDtypeStruct((_D0, _D1, _N), jnp.float32),
    scratch_types=[
        pltpu.VMEM((_EPW * _D1 + 16,), jnp.int32),
        pltpu.VMEM((_NB, _D1, _N), jnp.float32),
        pltpu.SemaphoreType.DMA((_NB,)),
    ],
    compiler_params=pltpu.CompilerParams(
        use_tc_tiling_on_sc=True,
        needs_layout_passes=False,
    ),
)
def _sc_onehot(idx_hbm, out_hbm, idx_v, buf, sems):
    wid = lax.axis_index("c") * 16 + lax.axis_index("s")
    lanes = lax.iota(jnp.int32, 16)
    ones = jnp.full((16,), 1.0, jnp.float32)
    zeros = jnp.zeros((16,), jnp.float32)
    tail_mask = lanes < (_D1 - 16)

    pltpu.sync_copy(
        idx_hbm.at[pl.ds(wid * _EPW * _D1, _EPW * _D1)],
        idx_v.at[pl.ds(0, _EPW * _D1)],
    )

    def _zero_row(r, _):
        s = r // _D1
        rr = r - s * _D1

        def _zero_vec(j, _):
            buf[s, rr, pl.ds(j * 16, 16)] = zeros
            return 0

        lax.fori_loop(0, _N // 16, _zero_vec, 0)
        buf[s, rr, pl.ds(_N - 16, 16)] = zeros
        return 0

    lax.fori_loop(0, _NB * _D1, _zero_row, 0)

    def _positions(c):
        base = c * _D1
        cols0 = idx_v[pl.ds(base, 16)]
        cols1 = idx_v[pl.ds(base + 16, 16)]
        return cols0, cols1

    def _chunk(c, _):
        s = lax.rem(c, _NB)
        d0 = wid * _EPW + c

        @pl.when(c >= _NB)
        def _recycle():
            pltpu.make_async_copy(
                buf.at[s], out_hbm.at[d0 - _NB], sems.at[s]
            ).wait()
            oc0, oc1 = _positions(c - _NB)
            plsc.store_scatter(buf.at[s], [lanes, oc0], zeros)
            plsc.store_scatter(
                buf.at[s], [16 + lanes, oc1], zeros, mask=tail_mask
            )

        cols0, cols1 = _positions(c)
        plsc.store_scatter(buf.at[s], [lanes, cols0], ones)
        plsc.store_scatter(
            buf.at[s], [16 + lanes, cols1], ones, mask=tail_mask
        )
        pltpu.make_async_copy(buf.at[s], out_hbm.at[d0], sems.at[s]).start()
        return 0

    lax.fori_loop(0, _EPW, _chunk, 0)

    for k in range(_NB):
        pltpu.make_async_copy(
            buf.at[k], out_hbm.at[wid * _EPW + _EPW - _NB + k], sems.at[k]
        ).wait()


def kernel(input):
    return _sc_onehot(input.reshape(-1))


# speedup vs baseline: 10.1187x; 5.1346x over previous
import jax
import jax.numpy as jnp
from jax.experimental import pallas as pl
from jax.experimental.pallas import tpu as pltpu

_N_CLASSES = 1000
_D1 = 26
_BB = 128


def _onehot_body(idx_ref, out_ref):
    idx = idx_ref[...]
    iota = jax.lax.broadcasted_iota(jnp.int32, (_D1, _N_CLASSES, _BB), 1)
    out_ref[...] = (iota == idx[:, None, :]).astype(jnp.float32)


def kernel(input):
    idx_t = input.T
    grid = 4096 // _BB
    out = pl.pallas_call(
        _onehot_body,
        grid=(grid,),
        in_specs=[pl.BlockSpec((_D1, _BB), lambda i: (0, i))],
        out_specs=pl.BlockSpec((_D1, _N_CLASSES, _BB), lambda i: (0, 0, i)),
        out_shape=jax.ShapeDtypeStruct((_D1, _N_CLASSES, 4096), jnp.float32),
        compiler_params=pltpu.CompilerParams(
            dimension_semantics=("arbitrary",),
        ),
    )(idx_t)
    return jnp.transpose(out, (2, 0, 1))
